# baseline (speedup 1.0000x reference)
import functools

import jax
import jax.numpy as jnp
from jax.experimental import pallas as pl
from jax.experimental.pallas import tpu as pltpu
from jax.experimental.pallas import tpu_sc as plsc


def _sc_gather(table, flat_idx, n_rows, row_width, window):
    mesh = plsc.VectorSubcoreMesh(core_axis_name="c", subcore_axis_name="s")
    n_steps = n_rows // window
    idx3d = flat_idx.reshape(n_steps, 1, window)

    @functools.partial(
        pl.kernel,
        out_type=jax.ShapeDtypeStruct((n_rows, row_width), table.dtype),
        mesh=mesh,
    )
    def k(x_hbm, i_hbm, o_hbm):
        def body(i_vmem, o_vmem):
            pltpu.sync_copy(x_hbm.at[i_vmem.at[0, 0]], o_vmem)

        pltpu.emit_pipeline(
            body,
            grid=(n_steps,),
            in_specs=[
                pl.BlockSpec((1, 1, window), index_map=lambda i: (i, 0, 0))
            ],
            out_specs=[
                pl.BlockSpec((window, row_width), index_map=lambda i: (i, 0))
            ],
            core_axis_name=("c", "s"),
            dimension_semantics=(pltpu.PARALLEL,),
        )(i_hbm, o_hbm)

    return k(table, idx3d)


def _tc_gemm(g, w_stack, b_stack, block_rows, blocks_per_batch, inner_blocks):
    n_rows, k_dim = g.shape
    n_out = w_stack.shape[-1]
    grid = n_rows // block_rows

    def body(g_ref, w_ref, b_ref, o_ref):
        o_ref[...] = (
            jnp.dot(g_ref[...], w_ref[0], preferred_element_type=jnp.float32)
            + b_ref[0]
        )

    def wsel(i):
        return jnp.where((i % blocks_per_batch) < inner_blocks, 0, 1)

    return pl.pallas_call(
        body,
        grid=(grid,),
        in_specs=[
            pl.BlockSpec((block_rows, k_dim), lambda i: (i, 0)),
            pl.BlockSpec((1, k_dim, n_out), lambda i: (wsel(i), 0, 0)),
            pl.BlockSpec((1, 1, n_out), lambda i: (wsel(i), 0, 0)),
        ],
        out_specs=pl.BlockSpec((block_rows, n_out), lambda i: (i, 0)),
        out_shape=jax.ShapeDtypeStruct((n_rows, n_out), jnp.float32),
    )(g, w_stack, b_stack)


def kernel(x, indices_inner, indices_outer, W_inner, b_inner, W_outer, b_outer):
    bs, n_nodes, cin = x.shape
    nb_in, seq = indices_inner.shape
    nb_out = indices_outer.shape[0]
    cout = W_inner.shape[0]

    table = jnp.concatenate(
        [x.reshape(bs * n_nodes, cin), jnp.zeros((8, cin), x.dtype)], axis=0
    )

    idx_cat = jnp.concatenate([indices_inner, indices_outer], axis=0)
    batch_off = (jnp.arange(bs, dtype=jnp.int32) * n_nodes)[:, None, None]
    gidx = jnp.where(idx_cat[None] < n_nodes, idx_cat[None] + batch_off,
                     bs * n_nodes)
    flat_idx = gidx.reshape(-1)

    n_rows = flat_idx.shape[0]
    gathered = _sc_gather(table, flat_idx, n_rows, cin, window=128)

    g2d = gathered.reshape(bs * (nb_in + nb_out), seq * cin)

    w_stack = jnp.stack([W_inner.T, W_outer.T], axis=0)
    b_stack = jnp.stack([b_inner, b_outer], axis=0)[:, None, :]

    block_rows = 1000
    out2d = _tc_gemm(
        g2d, w_stack, b_stack,
        block_rows=block_rows,
        blocks_per_batch=(nb_in + nb_out) // block_rows,
        inner_blocks=nb_in // block_rows,
    )
    return out2d.reshape(bs, nb_in + nb_out, cout)
